# Initial kernel scaffold; baseline (speedup 1.0000x reference)
import jax
import jax.numpy as jnp
from jax.experimental import pallas as pl


def kernel(positions, radii):
    raise NotImplementedError("write your pallas kernel here")


# speedup vs baseline: 1314.7785x; 1314.7785x over previous
import functools

import jax
import jax.numpy as jnp
import numpy as np
from jax.experimental import pallas as pl
from jax.experimental.pallas import tpu as pltpu

N = 2048
TR = 256
CHUNK = 256


@functools.lru_cache(maxsize=1)
def _constants():
    gate = np.asarray(jax.random.bernoulli(jax.random.key(1), 0.5, (N, N)))
    ii = np.arange(N)
    g = (gate & (ii[:, None] > ii[None, :])).astype(np.int8)
    row_keys = jax.random.split(jax.random.key(2), N)
    u = np.asarray(
        jax.vmap(lambda k: jax.random.uniform(k, (), jnp.float32))(row_keys))
    lt = np.triu(np.ones((CHUNK, CHUNK), np.float32))
    return g, u.reshape(N, 1), lt


_G_CONST, _U_CONST, _LT_CONST = _constants()


def _body(g_ref, pxr_ref, pyr_ref, rr_ref, pxc_ref, pyc_ref, rc_ref, u_ref,
          lt_ref, ox_ref, oy_ref, mask_scr):
    pxr = pxr_ref[...]
    pyr = pyr_ref[...]
    rr = rr_ref[...]
    pxc = pxc_ref[...]
    pyc = pyc_ref[...]
    rc = rc_ref[...]

    dx = pxr - pxc
    dy = pyr - pyc
    dist = jnp.sqrt(dx * dx + dy * dy + 1e-12)
    depth = (rc + rr) - dist
    maskf = ((depth > 0.0) & (g_ref[...] != 0)).astype(jnp.float32)
    mask_scr[...] = maskf

    cnt = jnp.sum(maskf, axis=1, keepdims=True)
    q = 1.0 / jnp.maximum(cnt, 1.0)
    cmax = jnp.max(cnt).astype(jnp.int32)

    def loop1(k, t):
        kf = k.astype(jnp.float32)
        return jnp.where(kf <= cnt, t + q, t)

    tc = jax.lax.fori_loop(1, cmax + 1, loop1, jnp.zeros_like(q))
    r = tc * (1.0 - u_ref[...])

    def loop2(k, carry):
        t, m = carry
        kf = k.astype(jnp.float32)
        live = kf <= cnt
        t2 = jnp.where(live, t + q, t)
        m2 = m + jnp.where(live & (t2 < r), 1.0, 0.0)
        return t2, m2

    _, m_ord = jax.lax.fori_loop(1, cmax + 1, loop2,
                                 (jnp.zeros_like(q), jnp.zeros_like(q)))
    target = m_ord + 1.0

    lt = lt_ref[...]
    accx = jnp.zeros_like(cnt)
    accy = jnp.zeros_like(cnt)
    base = jnp.zeros_like(cnt)
    for c in range(N // CHUNK):
        lo, hi = c * CHUNK, (c + 1) * CHUNK
        mc = mask_scr[:, lo:hi]
        pc = jax.lax.dot(mc, lt,
                         precision=jax.lax.Precision.HIGHEST) + base
        ind = ((pc == target) & (mc > 0.0)).astype(jnp.float32)
        dxc = pxr[:, lo:hi] - pxc
        dyc = pyr[:, lo:hi] - pyc
        distc = jnp.sqrt(dxc * dxc + dyc * dyc + 1e-12)
        depthc = (rc + rr[:, lo:hi]) - distc
        inv = ind * (depthc / distc)
        accx = accx + jnp.sum(dxc * inv, axis=1, keepdims=True)
        accy = accy + jnp.sum(dyc * inv, axis=1, keepdims=True)
        base = pc[:, CHUNK - 1:CHUNK]
    ox_ref[...] = accx
    oy_ref[...] = accy


def kernel(positions, radii):
    g = jnp.asarray(_G_CONST)
    u = jnp.asarray(_U_CONST)
    lt = jnp.asarray(_LT_CONST)
    px_row = positions[:, 0].reshape(1, N)
    py_row = positions[:, 1].reshape(1, N)
    r_row = radii.reshape(1, N)
    px_col = positions[:, 0].reshape(N, 1)
    py_col = positions[:, 1].reshape(N, 1)
    r_col = radii.reshape(N, 1)

    grid = (N // TR,)
    full_row = pl.BlockSpec((1, N), lambda t: (0, 0))
    tile_col = pl.BlockSpec((TR, 1), lambda t: (t, 0))
    out_x, out_y = pl.pallas_call(
        _body,
        grid=grid,
        in_specs=[
            pl.BlockSpec((TR, N), lambda t: (t, 0)),
            full_row, full_row, full_row,
            tile_col, tile_col, tile_col,
            tile_col,
            pl.BlockSpec((CHUNK, CHUNK), lambda t: (0, 0)),
        ],
        out_specs=[tile_col, tile_col],
        out_shape=[
            jax.ShapeDtypeStruct((N, 1), jnp.float32),
            jax.ShapeDtypeStruct((N, 1), jnp.float32),
        ],
        scratch_shapes=[pltpu.VMEM((TR, N), jnp.float32)],
    )(g, px_row, py_row, r_row, px_col, py_col, r_col, u, lt)
    return jnp.concatenate([out_x, out_y], axis=1)


# speedup vs baseline: 1856.2218x; 1.4118x over previous
import functools

import jax
import jax.numpy as jnp
import numpy as np
from jax.experimental import pallas as pl
from jax.experimental.pallas import tpu as pltpu

N = 2048
TR = 256
CHUNK = 256


@functools.lru_cache(maxsize=1)
def _constants():
    gate = np.asarray(jax.random.bernoulli(jax.random.key(1), 0.5, (N, N)))
    ii = np.arange(N)
    g = (gate & (ii[:, None] > ii[None, :])).astype(np.int8)
    row_keys = jax.random.split(jax.random.key(2), N)
    u = np.asarray(
        jax.vmap(lambda k: jax.random.uniform(k, (), jnp.float32))(row_keys))
    lt = np.triu(np.ones((CHUNK, CHUNK), np.float32))
    return g, u.reshape(N, 1), lt


_G_CONST, _U_CONST, _LT_CONST = _constants()


def _body(g_ref, pxr_ref, pyr_ref, rr_ref, pxc_ref, pyc_ref, rc_ref, u_ref,
          lt_ref, ox_ref, oy_ref, mask_scr, px_acc, py_acc, rad_acc,
          base_acc):
    pid = pl.program_id(0)
    pxr = pxr_ref[...]
    pyr = pyr_ref[...]
    rr = rr_ref[...]
    pxc = pxc_ref[...]
    pyc = pyc_ref[...]
    rc = rc_ref[...]
    ncols = (pid + 1) * TR

    for cb in range(N // CHUNK):
        lo, hi = cb * CHUNK, (cb + 1) * CHUNK

        @pl.when(lo < ncols)
        def _compute():
            dx = pxr[:, lo:hi] - pxc
            dy = pyr[:, lo:hi] - pyc
            dist = jnp.sqrt(dx * dx + dy * dy + 1e-12)
            depth = (rc + rr[:, lo:hi]) - dist
            mask_scr[:, lo:hi] = jnp.where(
                (depth > 0.0) & (g_ref[:, lo:hi] != 0), 1.0, 0.0)

        @pl.when(lo >= ncols)
        def _zero():
            mask_scr[:, lo:hi] = jnp.zeros((TR, CHUNK), jnp.float32)

    cnt = jnp.sum(mask_scr[...], axis=1, keepdims=True)
    q = 1.0 / jnp.maximum(cnt, 1.0)
    cmax = jnp.max(cnt).astype(jnp.int32)

    def loop1(k, t):
        kf = k.astype(jnp.float32)
        return jnp.where(kf <= cnt, t + q, t)

    tc = jax.lax.fori_loop(1, cmax + 1, loop1, jnp.zeros_like(q))
    r = tc * (1.0 - u_ref[...])

    def loop2(k, carry):
        t, m = carry
        kf = k.astype(jnp.float32)
        live = kf <= cnt
        t2 = jnp.where(live, t + q, t)
        m2 = m + jnp.where(live & (t2 < r), 1.0, 0.0)
        return t2, m2

    _, m_ord = jax.lax.fori_loop(1, cmax + 1, loop2,
                                 (jnp.zeros_like(q), jnp.zeros_like(q)))
    target = m_ord + 1.0

    lt = lt_ref[...]
    zero_col = jnp.zeros((TR, 1), jnp.float32)
    px_acc[...] = zero_col
    py_acc[...] = zero_col
    rad_acc[...] = zero_col
    base_acc[...] = zero_col
    for cb in range(N // CHUNK):
        lo, hi = cb * CHUNK, (cb + 1) * CHUNK

        @pl.when(lo < ncols)
        def _select():
            mc = mask_scr[:, lo:hi]
            pc = jax.lax.dot(mc, lt) + base_acc[...]
            ind = jnp.where(pc == target, mc, 0.0)
            px_acc[...] += jnp.sum(ind * pxr[:, lo:hi], axis=1, keepdims=True)
            py_acc[...] += jnp.sum(ind * pyr[:, lo:hi], axis=1, keepdims=True)
            rad_acc[...] += jnp.sum(ind * rr[:, lo:hi], axis=1, keepdims=True)
            base_acc[...] = pc[:, CHUNK - 1:CHUNK]

    jx = px_acc[...]
    jy = py_acc[...]
    jr = rad_acc[...]
    dxs = jx - pxc
    dys = jy - pyc
    dists = jnp.sqrt(dxs * dxs + dys * dys + 1e-12)
    depths = (rc + jr) - dists
    have = cnt > 0.0
    ox_ref[...] = jnp.where(have, (dxs / dists) * depths, 0.0)
    oy_ref[...] = jnp.where(have, (dys / dists) * depths, 0.0)


def kernel(positions, radii):
    g = jnp.asarray(_G_CONST)
    u = jnp.asarray(_U_CONST)
    lt = jnp.asarray(_LT_CONST)
    px_row = positions[:, 0].reshape(1, N)
    py_row = positions[:, 1].reshape(1, N)
    r_row = radii.reshape(1, N)
    px_col = positions[:, 0].reshape(N, 1)
    py_col = positions[:, 1].reshape(N, 1)
    r_col = radii.reshape(N, 1)

    grid = (N // TR,)
    full_row = pl.BlockSpec((1, N), lambda t: (0, 0))
    tile_col = pl.BlockSpec((TR, 1), lambda t: (t, 0))
    out_x, out_y = pl.pallas_call(
        _body,
        grid=grid,
        in_specs=[
            pl.BlockSpec((TR, N), lambda t: (t, 0)),
            full_row, full_row, full_row,
            tile_col, tile_col, tile_col,
            tile_col,
            pl.BlockSpec((CHUNK, CHUNK), lambda t: (0, 0)),
        ],
        out_specs=[tile_col, tile_col],
        out_shape=[
            jax.ShapeDtypeStruct((N, 1), jnp.float32),
            jax.ShapeDtypeStruct((N, 1), jnp.float32),
        ],
        scratch_shapes=[
            pltpu.VMEM((TR, N), jnp.float32),
            pltpu.VMEM((TR, 1), jnp.float32),
            pltpu.VMEM((TR, 1), jnp.float32),
            pltpu.VMEM((TR, 1), jnp.float32),
            pltpu.VMEM((TR, 1), jnp.float32),
        ],
    )(g, px_row, py_row, r_row, px_col, py_col, r_col, u, lt)
    return jnp.concatenate([out_x, out_y], axis=1)
